# Initial kernel scaffold; baseline (speedup 1.0000x reference)
import functools

import jax
import jax.numpy as jnp
from jax import lax
from jax.experimental import pallas as pl
from jax.experimental.pallas import tpu as pltpu
from jax.experimental.pallas import tpu_sc as plsc

N = 10000
E = 320000
D = 128
H = 256
C = 64
K = 10
ALPHA = 0.1

NPAD = 10240
SCRAP = N
NCORES = 2
NSUB = 16
NW = NCORES * NSUB
CHUNK = 128
CHUNKS = 79
EPAD = NW * CHUNKS * CHUNK
RPT = NPAD // NSUB

_SC_MESH = plsc.VectorSubcoreMesh(core_axis_name="c", subcore_axis_name="s")


def _mlp_body(x_ref, w1_ref, w2_ref, o_ref):
    hh = jnp.maximum(
        jnp.dot(x_ref[...], w1_ref[...], preferred_element_type=jnp.float32), 0.0)
    o_ref[...] = jnp.dot(hh, w2_ref[...], preferred_element_type=jnp.float32)


def _mlp(xp, W1, W2):
    R = 640
    return pl.pallas_call(
        _mlp_body,
        grid=(NPAD // R,),
        in_specs=[pl.BlockSpec((R, D), lambda i: (i, 0)),
                  pl.BlockSpec((D, H), lambda i: (0, 0)),
                  pl.BlockSpec((H, C), lambda i: (0, 0))],
        out_specs=pl.BlockSpec((R, C), lambda i: (i, 0)),
        out_shape=jax.ShapeDtypeStruct((NPAD, C), jnp.float32),
    )(xp, W1, W2)


def _deg_body(sidx_hbm, didx_hbm, ones_hbm, z16_hbm, outa_hbm, outb_hbm,
              sidx_v, didx_v, ones_v, dega, degb):
    c = lax.axis_index("c")
    s = lax.axis_index("s")
    wid = s * NCORES + c
    r0 = s * RPT
    pltpu.sync_copy(sidx_hbm.at[wid], sidx_v)
    pltpu.sync_copy(didx_hbm.at[wid], didx_v)
    pltpu.sync_copy(ones_hbm, ones_v)
    pltpu.sync_copy(z16_hbm.at[pl.ds(r0, RPT)], dega.at[pl.ds(r0, RPT)])
    pltpu.sync_copy(z16_hbm.at[pl.ds(r0, RPT)], degb.at[pl.ds(r0, RPT)])
    plsc.subcore_barrier()

    def body(j, carry):
        pltpu.sync_copy(ones_v, dega.at[sidx_v.at[j]], add=True)
        pltpu.sync_copy(ones_v, degb.at[didx_v.at[j]], add=True)
        return carry

    lax.fori_loop(0, CHUNKS, body, 0)
    plsc.subcore_barrier()
    pltpu.sync_copy(dega.at[pl.ds(r0, RPT)], outa_hbm.at[c, pl.ds(r0, RPT)])
    pltpu.sync_copy(degb.at[pl.ds(r0, RPT)], outb_hbm.at[c, pl.ds(r0, RPT)])


def _deg(sidx, didx, ones16, z16):
    f = pl.kernel(
        _deg_body,
        mesh=_SC_MESH,
        out_type=[jax.ShapeDtypeStruct((NCORES, NPAD, 16), jnp.float32),
                  jax.ShapeDtypeStruct((NCORES, NPAD, 16), jnp.float32)],
        scratch_types=[
            pltpu.VMEM((CHUNKS, CHUNK), jnp.int32),
            pltpu.VMEM((CHUNKS, CHUNK), jnp.int32),
            pltpu.VMEM((CHUNK, 16), jnp.float32),
            pltpu.VMEM_SHARED((NPAD, 16), jnp.float32),
            pltpu.VMEM_SHARED((NPAD, 16), jnp.float32),
        ],
    )
    return f(sidx, didx, ones16, z16)


def _step_body(u_hbm, z64_hbm, sidx_hbm, didx_hbm, out_hbm,
               sidx_v, didx_v, rows_v, sem, agg):
    c = lax.axis_index("c")
    s = lax.axis_index("s")
    wid = s * NCORES + c
    r0 = s * RPT
    pltpu.sync_copy(sidx_hbm.at[wid], sidx_v)
    pltpu.sync_copy(didx_hbm.at[wid], didx_v)
    pltpu.sync_copy(z64_hbm.at[pl.ds(r0, RPT)], agg.at[pl.ds(r0, RPT)])
    plsc.subcore_barrier()

    def body(j, carry):
        pltpu.async_copy(u_hbm.at[sidx_v.at[j]], rows_v, sem).wait()
        pltpu.sync_copy(rows_v, agg.at[didx_v.at[j]], add=True)
        return carry

    lax.fori_loop(0, CHUNKS, body, 0)
    plsc.subcore_barrier()
    pltpu.sync_copy(agg.at[pl.ds(r0, RPT)], out_hbm.at[c, pl.ds(r0, RPT)])


def _step(u, z64, sidx, didx):
    f = pl.kernel(
        _step_body,
        mesh=_SC_MESH,
        out_type=jax.ShapeDtypeStruct((NCORES, NPAD, C), jnp.float32),
        scratch_types=[
            pltpu.VMEM((CHUNKS, CHUNK), jnp.int32),
            pltpu.VMEM((CHUNKS, CHUNK), jnp.int32),
            pltpu.VMEM((CHUNK, C), jnp.float32),
            pltpu.SemaphoreType.DMA,
            pltpu.VMEM_SHARED((NPAD, C), jnp.float32),
        ],
    )
    return f(u, z64, sidx, didx)


def _prep_body(da_ref, db_ref, h0_ref, p_ref, b_ref, r_ref, u0_ref):
    i = pl.program_id(0)
    da = da_ref[...]
    db = db_ref[...]
    deg_out = jnp.maximum(da[0, :, 0:1] + da[1, :, 0:1], 1.0)
    deg_in = jnp.maximum(db[0, :, 0:1] + db[1, :, 0:1], 1.0)
    av = lax.rsqrt(deg_out)
    bv = lax.rsqrt(deg_in)
    rowid = i * 640 + lax.broadcasted_iota(jnp.int32, (640, 1), 0)
    valid = rowid < N
    p_ref[...] = jnp.where(valid, (1.0 - ALPHA) * av * bv, 0.0)
    b_ref[...] = bv
    h0 = h0_ref[...]
    r_ref[...] = (ALPHA * av) * h0
    u0_ref[...] = av * h0


def _prep(da, db, h0p):
    R = 640
    return pl.pallas_call(
        _prep_body,
        grid=(NPAD // R,),
        in_specs=[pl.BlockSpec((NCORES, R, 16), lambda i: (0, i, 0)),
                  pl.BlockSpec((NCORES, R, 16), lambda i: (0, i, 0)),
                  pl.BlockSpec((R, C), lambda i: (i, 0))],
        out_specs=[pl.BlockSpec((R, 1), lambda i: (i, 0)),
                   pl.BlockSpec((R, 1), lambda i: (i, 0)),
                   pl.BlockSpec((R, C), lambda i: (i, 0)),
                   pl.BlockSpec((R, C), lambda i: (i, 0))],
        out_shape=[jax.ShapeDtypeStruct((NPAD, 1), jnp.float32),
                   jax.ShapeDtypeStruct((NPAD, 1), jnp.float32),
                   jax.ShapeDtypeStruct((NPAD, C), jnp.float32),
                   jax.ShapeDtypeStruct((NPAD, C), jnp.float32)],
    )(da, db, h0p)


def _upd_body(agg_ref, p_ref, r_ref, u_ref):
    ag = agg_ref[...]
    u_ref[...] = p_ref[...] * (ag[0] + ag[1]) + r_ref[...]


def _upd(aggp, p, r):
    R = 640
    return pl.pallas_call(
        _upd_body,
        grid=(NPAD // R,),
        in_specs=[pl.BlockSpec((NCORES, R, C), lambda i: (0, i, 0)),
                  pl.BlockSpec((R, 1), lambda i: (i, 0)),
                  pl.BlockSpec((R, C), lambda i: (i, 0))],
        out_specs=pl.BlockSpec((R, C), lambda i: (i, 0)),
        out_shape=jax.ShapeDtypeStruct((NPAD, C), jnp.float32),
    )(aggp, p, r)


def _fin_body(agg_ref, b_ref, h0_ref, h_ref):
    ag = agg_ref[...]
    h_ref[...] = ((1.0 - ALPHA) * b_ref[...] * (ag[0] + ag[1])
                  + ALPHA * h0_ref[...])


def _fin(aggp, b, h0p):
    R = 500
    return pl.pallas_call(
        _fin_body,
        grid=(N // R,),
        in_specs=[pl.BlockSpec((NCORES, R, C), lambda i: (0, i, 0)),
                  pl.BlockSpec((R, 1), lambda i: (i, 0)),
                  pl.BlockSpec((R, C), lambda i: (i, 0))],
        out_specs=pl.BlockSpec((R, C), lambda i: (i, 0)),
        out_shape=jax.ShapeDtypeStruct((N, C), jnp.float32),
    )(aggp, b, h0p)


def kernel(features, edge_index, W1, W2):
    src = edge_index[0]
    dst = edge_index[1]
    pad = jnp.full((EPAD - E,), SCRAP, jnp.int32)
    sidx = jnp.concatenate([src, pad]).reshape(NW, CHUNKS, CHUNK)
    didx = jnp.concatenate([dst, pad]).reshape(NW, CHUNKS, CHUNK)
    xp = jnp.pad(features, ((0, NPAD - N), (0, 0)))

    h0p = _mlp(xp, W1, W2)

    ones16 = jnp.ones((CHUNK, 16), jnp.float32)
    z16 = jnp.zeros((NPAD, 16), jnp.float32)
    z64 = jnp.zeros((NPAD, C), jnp.float32)

    da, db = _deg(sidx, didx, ones16, z16)
    p, b, r, u0 = _prep(da, db, h0p)

    u = u0
    for _ in range(K - 1):
        aggp = _step(u, z64, sidx, didx)
        u = _upd(aggp, p, r)
    aggp = _step(u, z64, sidx, didx)
    return _fin(aggp, b, h0p)


# speedup vs baseline: 8.8066x; 8.8066x over previous
import functools

import jax
import jax.numpy as jnp
from jax import lax
from jax.experimental import pallas as pl
from jax.experimental.pallas import tpu as pltpu
from jax.experimental.pallas import tpu_sc as plsc

N = 10000
E = 320000
D = 128
H = 256
C = 64
K = 10
ALPHA = 0.1

NPAD = 10240
SCRAP = N
NCORES = 2
NSUB = 16
NW = NCORES * NSUB
CHUNK = 128
CHUNKS = 79
EPAD = NW * CHUNKS * CHUNK
RPT = NPAD // NSUB

_SC_MESH = plsc.VectorSubcoreMesh(core_axis_name="c", subcore_axis_name="s")
_SC_PARAMS = pltpu.CompilerParams(use_tc_tiling_on_sc=False)


def _mlp_body(x_ref, w1_ref, w2_ref, o_ref):
    hh = jnp.maximum(
        jnp.dot(x_ref[...], w1_ref[...], preferred_element_type=jnp.float32), 0.0)
    o_ref[...] = jnp.dot(hh, w2_ref[...], preferred_element_type=jnp.float32)


def _mlp(xp, W1, W2):
    R = 640
    return pl.pallas_call(
        _mlp_body,
        grid=(NPAD // R,),
        in_specs=[pl.BlockSpec((R, D), lambda i: (i, 0)),
                  pl.BlockSpec((D, H), lambda i: (0, 0)),
                  pl.BlockSpec((H, C), lambda i: (0, 0))],
        out_specs=pl.BlockSpec((R, C), lambda i: (i, 0)),
        out_shape=jax.ShapeDtypeStruct((NPAD, C), jnp.float32),
    )(xp, W1, W2)


def _deg_body(sidx_hbm, didx_hbm, ones_hbm, z16_hbm, outa_hbm, outb_hbm,
              sidx_v, didx_v, ones_v, dega, degb):
    c = lax.axis_index("c")
    s = lax.axis_index("s")
    wid = s * NCORES + c
    r0 = s * RPT
    pltpu.sync_copy(sidx_hbm.at[wid], sidx_v)
    pltpu.sync_copy(didx_hbm.at[wid], didx_v)
    pltpu.sync_copy(ones_hbm, ones_v)
    pltpu.sync_copy(z16_hbm.at[pl.ds(r0, RPT)], dega.at[pl.ds(r0, RPT)])
    pltpu.sync_copy(z16_hbm.at[pl.ds(r0, RPT)], degb.at[pl.ds(r0, RPT)])
    plsc.subcore_barrier()

    def body(j, carry):
        pltpu.sync_copy(ones_v, dega.at[sidx_v.at[j]], add=True)
        pltpu.sync_copy(ones_v, degb.at[didx_v.at[j]], add=True)
        return carry

    lax.fori_loop(0, CHUNKS, body, 0)
    plsc.subcore_barrier()
    pltpu.sync_copy(dega.at[pl.ds(r0, RPT)], outa_hbm.at[c, pl.ds(r0, RPT)])
    pltpu.sync_copy(degb.at[pl.ds(r0, RPT)], outb_hbm.at[c, pl.ds(r0, RPT)])


def _deg(sidx, didx, ones16, z16):
    f = pl.kernel(
        _deg_body,
        mesh=_SC_MESH,
        out_type=[jax.ShapeDtypeStruct((NCORES, NPAD, 16), jnp.float32),
                  jax.ShapeDtypeStruct((NCORES, NPAD, 16), jnp.float32)],
        scratch_types=[
            pltpu.VMEM((CHUNKS, CHUNK), jnp.int32),
            pltpu.VMEM((CHUNKS, CHUNK), jnp.int32),
            pltpu.VMEM((CHUNK, 16), jnp.float32),
            pltpu.VMEM_SHARED((NPAD, 16), jnp.float32),
            pltpu.VMEM_SHARED((NPAD, 16), jnp.float32),
        ],
        compiler_params=_SC_PARAMS,
    )
    return f(sidx, didx, ones16, z16)


def _step_body(u_hbm, z64_hbm, sidx_hbm, didx_hbm, out_hbm,
               sidx_v, didx_v, rows_v, sem, agg):
    c = lax.axis_index("c")
    s = lax.axis_index("s")
    wid = s * NCORES + c
    r0 = s * RPT
    pltpu.sync_copy(sidx_hbm.at[wid], sidx_v)
    pltpu.sync_copy(didx_hbm.at[wid], didx_v)
    pltpu.sync_copy(z64_hbm.at[pl.ds(r0, RPT)], agg.at[pl.ds(r0, RPT)])
    plsc.subcore_barrier()

    def body(j, carry):
        pltpu.async_copy(u_hbm.at[sidx_v.at[j]], rows_v, sem).wait()
        pltpu.sync_copy(rows_v, agg.at[didx_v.at[j]], add=True)
        return carry

    lax.fori_loop(0, CHUNKS, body, 0)
    plsc.subcore_barrier()
    pltpu.sync_copy(agg.at[pl.ds(r0, RPT)], out_hbm.at[c, pl.ds(r0, RPT)])


def _step(u, z64, sidx, didx):
    f = pl.kernel(
        _step_body,
        mesh=_SC_MESH,
        out_type=jax.ShapeDtypeStruct((NCORES, NPAD, C), jnp.float32),
        scratch_types=[
            pltpu.VMEM((CHUNKS, CHUNK), jnp.int32),
            pltpu.VMEM((CHUNKS, CHUNK), jnp.int32),
            pltpu.VMEM((CHUNK, C), jnp.float32),
            pltpu.SemaphoreType.DMA,
            pltpu.VMEM_SHARED((NPAD, C), jnp.float32),
        ],
        compiler_params=_SC_PARAMS,
    )
    return f(u, z64, sidx, didx)


def _prep_body(da_ref, db_ref, h0_ref, p_ref, b_ref, r_ref, u0_ref):
    i = pl.program_id(0)
    da = da_ref[...]
    db = db_ref[...]
    deg_out = jnp.maximum(da[0, :, 0:1] + da[1, :, 0:1], 1.0)
    deg_in = jnp.maximum(db[0, :, 0:1] + db[1, :, 0:1], 1.0)
    av = lax.rsqrt(deg_out)
    bv = lax.rsqrt(deg_in)
    rowid = i * 640 + lax.broadcasted_iota(jnp.int32, (640, 1), 0)
    valid = rowid < N
    p_ref[...] = jnp.where(valid, (1.0 - ALPHA) * av * bv, 0.0)
    b_ref[...] = bv
    h0 = h0_ref[...]
    r_ref[...] = (ALPHA * av) * h0
    u0_ref[...] = av * h0


def _prep(da, db, h0p):
    R = 640
    return pl.pallas_call(
        _prep_body,
        grid=(NPAD // R,),
        in_specs=[pl.BlockSpec((NCORES, R, 16), lambda i: (0, i, 0)),
                  pl.BlockSpec((NCORES, R, 16), lambda i: (0, i, 0)),
                  pl.BlockSpec((R, C), lambda i: (i, 0))],
        out_specs=[pl.BlockSpec((R, 1), lambda i: (i, 0)),
                   pl.BlockSpec((R, 1), lambda i: (i, 0)),
                   pl.BlockSpec((R, C), lambda i: (i, 0)),
                   pl.BlockSpec((R, C), lambda i: (i, 0))],
        out_shape=[jax.ShapeDtypeStruct((NPAD, 1), jnp.float32),
                   jax.ShapeDtypeStruct((NPAD, 1), jnp.float32),
                   jax.ShapeDtypeStruct((NPAD, C), jnp.float32),
                   jax.ShapeDtypeStruct((NPAD, C), jnp.float32)],
    )(da, db, h0p)


def _upd_body(agg_ref, p_ref, r_ref, u_ref):
    ag = agg_ref[...]
    u_ref[...] = p_ref[...] * (ag[0] + ag[1]) + r_ref[...]


def _upd(aggp, p, r):
    R = 640
    return pl.pallas_call(
        _upd_body,
        grid=(NPAD // R,),
        in_specs=[pl.BlockSpec((NCORES, R, C), lambda i: (0, i, 0)),
                  pl.BlockSpec((R, 1), lambda i: (i, 0)),
                  pl.BlockSpec((R, C), lambda i: (i, 0))],
        out_specs=pl.BlockSpec((R, C), lambda i: (i, 0)),
        out_shape=jax.ShapeDtypeStruct((NPAD, C), jnp.float32),
    )(aggp, p, r)


def _fin_body(agg_ref, b_ref, h0_ref, h_ref):
    ag = agg_ref[...]
    h_ref[...] = ((1.0 - ALPHA) * b_ref[...] * (ag[0] + ag[1])
                  + ALPHA * h0_ref[...])


def _fin(aggp, b, h0p):
    R = 400
    return pl.pallas_call(
        _fin_body,
        grid=(N // R,),
        in_specs=[pl.BlockSpec((NCORES, R, C), lambda i: (0, i, 0)),
                  pl.BlockSpec((R, 1), lambda i: (i, 0)),
                  pl.BlockSpec((R, C), lambda i: (i, 0))],
        out_specs=pl.BlockSpec((R, C), lambda i: (i, 0)),
        out_shape=jax.ShapeDtypeStruct((N, C), jnp.float32),
    )(aggp, b, h0p)


def kernel(features, edge_index, W1, W2):
    src = edge_index[0]
    dst = edge_index[1]
    pad = jnp.full((EPAD - E,), SCRAP, jnp.int32)
    sidx = jnp.concatenate([src, pad]).reshape(NW, CHUNKS, CHUNK)
    didx = jnp.concatenate([dst, pad]).reshape(NW, CHUNKS, CHUNK)
    xp = jnp.pad(features, ((0, NPAD - N), (0, 0)))

    h0p = _mlp(xp, W1, W2)

    ones16 = jnp.ones((CHUNK, 16), jnp.float32)
    z16 = jnp.zeros((NPAD, 16), jnp.float32)
    z64 = jnp.zeros((NPAD, C), jnp.float32)

    da, db = _deg(sidx, didx, ones16, z16)
    p, b, r, u0 = _prep(da, db, h0p)

    u = u0
    for _ in range(K - 1):
        aggp = _step(u, z64, sidx, didx)
        u = _upd(aggp, p, r)
    aggp = _step(u, z64, sidx, didx)
    return _fin(aggp, b, h0p)
